# baseline (speedup 1.0000x reference)
import jax
import jax.numpy as jnp
from jax import lax
from jax.experimental import pallas as pl
from jax.experimental.pallas import tpu as pltpu
from jax.experimental.pallas import tpu_sc as plsc

CHARGE_DIM = 128
BATCH = 16
TOTAL_NODES = 32768

NUM_CORES = 2
NUM_SUBCORES = 16
LANES = 16
NUM_WORKERS = NUM_CORES * NUM_SUBCORES
ROWS_PER_WORKER = TOTAL_NODES // NUM_WORKERS
HEAD_ROWS = 16
CHUNK_ROWS = 112
NUM_CHUNKS = (ROWS_PER_WORKER - HEAD_ROWS) // CHUNK_ROWS
assert HEAD_ROWS + NUM_CHUNKS * CHUNK_ROWS == ROWS_PER_WORKER
COLS = CHARGE_DIM // LANES
CHUNK_ELEMS = CHUNK_ROWS * CHARGE_DIM


def _sc_body(charge_hbm, seg_hbm, out_hbm, charge_v, seg_v, bufh, buf0,
             buf1, semc, sems_, semh, sem0, sem1):
    wid = lax.axis_index("s") * NUM_CORES + lax.axis_index("c")
    base = wid * ROWS_PER_WORKER

    in0 = pltpu.async_copy(charge_hbm, charge_v, semc)
    in1 = pltpu.async_copy(seg_hbm.at[pl.ds(base, ROWS_PER_WORKER)],
                           seg_v.at[pl.ds(0, ROWS_PER_WORKER)], sems_)
    in0.wait()
    in1.wait()

    charge_reg = charge_v[...]
    zeros16 = jnp.zeros((LANES,), jnp.int32)

    def fast_fill(buf, cval, lo, hi):
        row = jnp.full((LANES,), cval, dtype=jnp.float32)

        def body(i, _):
            for j in range(COLS):
                buf[pl.ds(i * CHARGE_DIM + j * LANES, LANES)] = row
            return ()

        lax.fori_loop(lo, hi, body, (), unroll=2)

    def perrow_fill(buf, row0, lo, hi):
        def body(i, _):
            sv = seg_v[pl.ds(row0 + i, LANES)]
            sid = sv.at[zeros16].get(mode="promise_in_bounds")
            row = charge_reg.at[sid].get(mode="promise_in_bounds")
            for j in range(COLS):
                buf[pl.ds(i * CHARGE_DIM + j * LANES, LANES)] = row
            return ()

        lax.fori_loop(lo, hi, body, ())

    svh = seg_v[pl.ds(0, LANES)]
    uniform_h = svh[0] == svh[LANES - 1]
    cval_h = charge_reg.at[svh].get(mode="promise_in_bounds")[0]

    @pl.when(uniform_h)
    def _():
        fast_fill(bufh, cval_h, 0, HEAD_ROWS)

    @pl.when(jnp.logical_not(uniform_h))
    def _():
        perrow_fill(bufh, 0, 0, HEAD_ROWS)

    head_copy = pltpu.async_copy(
        bufh, out_hbm.at[pl.ds(base * CHARGE_DIM, HEAD_ROWS * CHARGE_DIM)],
        semh)

    def handle(buf, sem, c, row0, uniform, cval, same):
        @pl.when(c >= 2)
        def _():
            pltpu.make_async_copy(
                out_hbm.at[pl.ds(0, CHUNK_ELEMS)], buf, sem).wait()

        half = CHUNK_ROWS // 2
        helems = half * CHARGE_DIM
        dst0 = (base + row0) * CHARGE_DIM

        @pl.when(uniform & jnp.logical_not(same))
        def _():
            fast_fill(buf, cval, 0, half)
            pltpu.async_copy(buf.at[pl.ds(0, helems)],
                             out_hbm.at[pl.ds(dst0, helems)], sem)
            fast_fill(buf, cval, half, CHUNK_ROWS)
            pltpu.async_copy(buf.at[pl.ds(helems, helems)],
                             out_hbm.at[pl.ds(dst0 + helems, helems)], sem)

        @pl.when(jnp.logical_not(uniform))
        def _():
            perrow_fill(buf, row0, 0, half)
            pltpu.async_copy(buf.at[pl.ds(0, helems)],
                             out_hbm.at[pl.ds(dst0, helems)], sem)
            perrow_fill(buf, row0, half, CHUNK_ROWS)
            pltpu.async_copy(buf.at[pl.ds(helems, helems)],
                             out_hbm.at[pl.ds(dst0 + helems, helems)], sem)

        @pl.when(uniform & same)
        def _():
            pltpu.async_copy(
                buf, out_hbm.at[pl.ds(dst0, CHUNK_ELEMS)], sem)

    def chunk_body(c, carry):
        valid0, val0, valid1, val1 = carry
        row0 = HEAD_ROWS + c * CHUNK_ROWS
        par0 = lax.rem(c, 2) == 0
        sv0 = seg_v[pl.ds(row0, LANES)]
        svl = seg_v[pl.ds(row0 + CHUNK_ROWS - LANES, LANES)]
        uniform = sv0[0] == svl[LANES - 1]
        cval = charge_reg.at[sv0].get(mode="promise_in_bounds")[0]
        same0 = valid0 & (cval == val0)
        same1 = valid1 & (cval == val1)

        @pl.when(par0)
        def _():
            handle(buf0, sem0, c, row0, uniform, cval, same0)

        @pl.when(jnp.logical_not(par0))
        def _():
            handle(buf1, sem1, c, row0, uniform, cval, same1)

        valid0 = jnp.where(par0, uniform, valid0)
        val0 = jnp.where(par0, cval, val0)
        valid1 = jnp.where(par0, valid1, uniform)
        val1 = jnp.where(par0, val1, cval)
        return valid0, val0, valid1, val1

    lax.fori_loop(0, NUM_CHUNKS, chunk_body,
                  (jnp.bool_(False), jnp.float32(0.0),
                   jnp.bool_(False), jnp.float32(0.0)))

    pltpu.make_async_copy(
        out_hbm.at[pl.ds(0, CHUNK_ELEMS)], buf0, sem0).wait()
    pltpu.make_async_copy(
        out_hbm.at[pl.ds(0, CHUNK_ELEMS)], buf1, sem1).wait()
    head_copy.wait()


_sc_kernel = pl.kernel(
    _sc_body,
    out_type=jax.ShapeDtypeStruct((TOTAL_NODES * CHARGE_DIM,), jnp.float32),
    mesh=plsc.VectorSubcoreMesh(core_axis_name="c", subcore_axis_name="s"),
    scratch_types=[
        pltpu.VMEM((BATCH,), jnp.float32),
        pltpu.VMEM((ROWS_PER_WORKER + LANES,), jnp.int32),
        pltpu.VMEM((HEAD_ROWS * CHARGE_DIM,), jnp.float32),
        pltpu.VMEM((CHUNK_ELEMS,), jnp.float32),
        pltpu.VMEM((CHUNK_ELEMS,), jnp.float32),
        pltpu.SemaphoreType.DMA,
        pltpu.SemaphoreType.DMA,
        pltpu.SemaphoreType.DMA,
        pltpu.SemaphoreType.DMA,
        pltpu.SemaphoreType.DMA,
    ],
)


def kernel(charge, segment_ids):
    seg = segment_ids.astype(jnp.int32)
    out = _sc_kernel(charge.astype(jnp.float32), seg)
    return out.reshape(TOTAL_NODES, CHARGE_DIM)


# speedup vs baseline: 1.0484x; 1.0484x over previous
import jax
import jax.numpy as jnp
from jax import lax
from jax.experimental import pallas as pl
from jax.experimental.pallas import tpu as pltpu
from jax.experimental.pallas import tpu_sc as plsc

CHARGE_DIM = 128
BATCH = 16
TOTAL_NODES = 32768

NUM_CORES = 2
NUM_SUBCORES = 16
LANES = 16
NUM_WORKERS = NUM_CORES * NUM_SUBCORES
ROWS_PER_WORKER = TOTAL_NODES // NUM_WORKERS
HEAD_ROWS = 16
CHUNK_ROWS = 112
NUM_CHUNKS = (ROWS_PER_WORKER - HEAD_ROWS) // CHUNK_ROWS
assert HEAD_ROWS + NUM_CHUNKS * CHUNK_ROWS == ROWS_PER_WORKER
COLS = CHARGE_DIM // LANES
CHUNK_ELEMS = CHUNK_ROWS * CHARGE_DIM


def _sc_body(charge_hbm, seg_hbm, out_hbm, charge_v, seg_v, bufh, buf,
             semc, sems_, semh, sem):
    wid = lax.axis_index("s") * NUM_CORES + lax.axis_index("c")
    base = wid * ROWS_PER_WORKER

    in0 = pltpu.async_copy(charge_hbm, charge_v, semc)
    in1 = pltpu.async_copy(seg_hbm.at[pl.ds(base, ROWS_PER_WORKER)],
                           seg_v.at[pl.ds(0, ROWS_PER_WORKER)], sems_)
    in0.wait()
    in1.wait()

    charge_reg = charge_v[...]
    zeros16 = jnp.zeros((LANES,), jnp.int32)

    def fast_fill(buf_, off, cval, rows):
        row = jnp.full((LANES,), cval, dtype=jnp.float32)

        def body(i, _):
            for j in range(COLS):
                buf_[pl.ds(off + i * CHARGE_DIM + j * LANES, LANES)] = row
            return ()

        lax.fori_loop(0, rows, body, (), unroll=2)

    def perrow_fill(buf_, off, row0, rows):
        def body(i, _):
            sv = seg_v[pl.ds(row0 + i, LANES)]
            sid = sv.at[zeros16].get(mode="promise_in_bounds")
            row = charge_reg.at[sid].get(mode="promise_in_bounds")
            for j in range(COLS):
                buf_[pl.ds(off + i * CHARGE_DIM + j * LANES, LANES)] = row
            return ()

        lax.fori_loop(0, rows, body, ())

    svh = seg_v[pl.ds(0, LANES)]
    uniform_h = svh[0] == svh[LANES - 1]
    cval_h = charge_reg.at[svh].get(mode="promise_in_bounds")[0]

    @pl.when(uniform_h)
    def _():
        fast_fill(bufh, 0, cval_h, HEAD_ROWS)

    @pl.when(jnp.logical_not(uniform_h))
    def _():
        perrow_fill(bufh, 0, 0, HEAD_ROWS)

    head_copy = pltpu.async_copy(
        bufh, out_hbm.at[pl.ds(base * CHARGE_DIM, HEAD_ROWS * CHARGE_DIM)],
        semh)

    def chunk_body(c, carry):
        valid0, val0, valid1, val1 = carry
        row0 = HEAD_ROWS + c * CHUNK_ROWS
        par = lax.rem(c, 2)
        par0 = par == 0
        off = par * CHUNK_ELEMS
        sv0 = seg_v[pl.ds(row0, LANES)]
        svl = seg_v[pl.ds(row0 + CHUNK_ROWS - LANES, LANES)]
        uniform = sv0[0] == svl[LANES - 1]
        cval = charge_reg.at[sv0].get(mode="promise_in_bounds")[0]
        same = jnp.where(par0, valid0 & (cval == val0),
                         valid1 & (cval == val1))

        @pl.when(c >= 2)
        def _():
            pltpu.make_async_copy(
                out_hbm.at[pl.ds(0, CHUNK_ELEMS)],
                buf.at[pl.ds(off, CHUNK_ELEMS)], sem.at[par]).wait()

        @pl.when(uniform & jnp.logical_not(same))
        def _():
            fast_fill(buf, off, cval, CHUNK_ROWS)

        @pl.when(jnp.logical_not(uniform))
        def _():
            perrow_fill(buf, off, row0, CHUNK_ROWS)

        pltpu.async_copy(
            buf.at[pl.ds(off, CHUNK_ELEMS)],
            out_hbm.at[pl.ds((base + row0) * CHARGE_DIM, CHUNK_ELEMS)],
            sem.at[par])

        valid0 = jnp.where(par0, uniform, valid0)
        val0 = jnp.where(par0, cval, val0)
        valid1 = jnp.where(par0, valid1, uniform)
        val1 = jnp.where(par0, val1, cval)
        return valid0, val0, valid1, val1

    lax.fori_loop(0, NUM_CHUNKS, chunk_body,
                  (jnp.bool_(False), jnp.float32(0.0),
                   jnp.bool_(False), jnp.float32(0.0)))

    pltpu.make_async_copy(
        out_hbm.at[pl.ds(0, CHUNK_ELEMS)],
        buf.at[pl.ds(0, CHUNK_ELEMS)], sem.at[0]).wait()
    pltpu.make_async_copy(
        out_hbm.at[pl.ds(0, CHUNK_ELEMS)],
        buf.at[pl.ds(CHUNK_ELEMS, CHUNK_ELEMS)], sem.at[1]).wait()
    head_copy.wait()


_sc_kernel = pl.kernel(
    _sc_body,
    out_type=jax.ShapeDtypeStruct((TOTAL_NODES * CHARGE_DIM,), jnp.float32),
    mesh=plsc.VectorSubcoreMesh(core_axis_name="c", subcore_axis_name="s"),
    scratch_types=[
        pltpu.VMEM((BATCH,), jnp.float32),
        pltpu.VMEM((ROWS_PER_WORKER + LANES,), jnp.int32),
        pltpu.VMEM((HEAD_ROWS * CHARGE_DIM,), jnp.float32),
        pltpu.VMEM((2 * CHUNK_ELEMS,), jnp.float32),
        pltpu.SemaphoreType.DMA,
        pltpu.SemaphoreType.DMA,
        pltpu.SemaphoreType.DMA,
        pltpu.SemaphoreType.DMA((2,)),
    ],
)


def kernel(charge, segment_ids):
    seg = segment_ids.astype(jnp.int32)
    out = _sc_kernel(charge.astype(jnp.float32), seg)
    return out.reshape(TOTAL_NODES, CHARGE_DIM)


# speedup vs baseline: 1.0542x; 1.0056x over previous
import jax
import jax.numpy as jnp
from jax import lax
from jax.experimental import pallas as pl
from jax.experimental.pallas import tpu as pltpu
from jax.experimental.pallas import tpu_sc as plsc

CHARGE_DIM = 128
BATCH = 16
TOTAL_NODES = 32768

NUM_CORES = 2
NUM_SUBCORES = 16
LANES = 16
NUM_WORKERS = NUM_CORES * NUM_SUBCORES
ROWS_PER_WORKER = TOTAL_NODES // NUM_WORKERS
HEAD_ROWS = 16
CHUNK_ROWS = 112
NUM_CHUNKS = (ROWS_PER_WORKER - HEAD_ROWS) // CHUNK_ROWS
assert HEAD_ROWS + NUM_CHUNKS * CHUNK_ROWS == ROWS_PER_WORKER
COLS = CHARGE_DIM // LANES
CHUNK_ELEMS = CHUNK_ROWS * CHARGE_DIM


def _sc_body(charge_hbm, seg_hbm, out_hbm, charge_v, seg_v, bufh, buf,
             semc, sems_, semh, sem):
    wid = lax.axis_index("s") * NUM_CORES + lax.axis_index("c")
    base = wid * ROWS_PER_WORKER

    in0 = pltpu.async_copy(charge_hbm, charge_v, semc)
    in1 = pltpu.async_copy(seg_hbm.at[pl.ds(base, ROWS_PER_WORKER)],
                           seg_v.at[pl.ds(0, ROWS_PER_WORKER)], sems_)
    in0.wait()
    in1.wait()

    charge_reg = charge_v[...]
    zeros16 = jnp.zeros((LANES,), jnp.int32)

    def fast_fill(buf_, off, cval, rows):
        row = jnp.full((LANES,), cval, dtype=jnp.float32)

        def body(i, _):
            for j in range(COLS):
                buf_[pl.ds(off + i * CHARGE_DIM + j * LANES, LANES)] = row
            return ()

        lax.fori_loop(0, rows, body, (), unroll=4)

    def perrow_fill(buf_, off, row0, rows):
        def body(i, _):
            sv = seg_v[pl.ds(row0 + i, LANES)]
            sid = sv.at[zeros16].get(mode="promise_in_bounds")
            row = charge_reg.at[sid].get(mode="promise_in_bounds")
            for j in range(COLS):
                buf_[pl.ds(off + i * CHARGE_DIM + j * LANES, LANES)] = row
            return ()

        lax.fori_loop(0, rows, body, ())

    svh = seg_v[pl.ds(0, LANES)]
    uniform_h = svh[0] == svh[LANES - 1]
    cval_h = charge_reg.at[svh].get(mode="promise_in_bounds")[0]

    @pl.when(uniform_h)
    def _():
        fast_fill(bufh, 0, cval_h, HEAD_ROWS)

    @pl.when(jnp.logical_not(uniform_h))
    def _():
        perrow_fill(bufh, 0, 0, HEAD_ROWS)

    head_copy = pltpu.async_copy(
        bufh, out_hbm.at[pl.ds(base * CHARGE_DIM, HEAD_ROWS * CHARGE_DIM)],
        semh)

    def chunk_body(c, carry):
        valid0, val0, valid1, val1 = carry
        row0 = HEAD_ROWS + c * CHUNK_ROWS
        par = lax.rem(c, 2)
        par0 = par == 0
        off = par * CHUNK_ELEMS
        sv0 = seg_v[pl.ds(row0, LANES)]
        svl = seg_v[pl.ds(row0 + CHUNK_ROWS - LANES, LANES)]
        uniform = sv0[0] == svl[LANES - 1]
        cval = charge_reg.at[sv0].get(mode="promise_in_bounds")[0]
        same = jnp.where(par0, valid0 & (cval == val0),
                         valid1 & (cval == val1))

        @pl.when(c >= 2)
        def _():
            pltpu.make_async_copy(
                out_hbm.at[pl.ds(0, CHUNK_ELEMS)],
                buf.at[pl.ds(off, CHUNK_ELEMS)], sem.at[par]).wait()

        @pl.when(uniform & jnp.logical_not(same))
        def _():
            fast_fill(buf, off, cval, CHUNK_ROWS)

        @pl.when(jnp.logical_not(uniform))
        def _():
            perrow_fill(buf, off, row0, CHUNK_ROWS)

        pltpu.async_copy(
            buf.at[pl.ds(off, CHUNK_ELEMS)],
            out_hbm.at[pl.ds((base + row0) * CHARGE_DIM, CHUNK_ELEMS)],
            sem.at[par])

        valid0 = jnp.where(par0, uniform, valid0)
        val0 = jnp.where(par0, cval, val0)
        valid1 = jnp.where(par0, valid1, uniform)
        val1 = jnp.where(par0, val1, cval)
        return valid0, val0, valid1, val1

    lax.fori_loop(0, NUM_CHUNKS, chunk_body,
                  (jnp.bool_(False), jnp.float32(0.0),
                   jnp.bool_(False), jnp.float32(0.0)))

    pltpu.make_async_copy(
        out_hbm.at[pl.ds(0, CHUNK_ELEMS)],
        buf.at[pl.ds(0, CHUNK_ELEMS)], sem.at[0]).wait()
    pltpu.make_async_copy(
        out_hbm.at[pl.ds(0, CHUNK_ELEMS)],
        buf.at[pl.ds(CHUNK_ELEMS, CHUNK_ELEMS)], sem.at[1]).wait()
    head_copy.wait()


_sc_kernel = pl.kernel(
    _sc_body,
    out_type=jax.ShapeDtypeStruct((TOTAL_NODES * CHARGE_DIM,), jnp.float32),
    mesh=plsc.VectorSubcoreMesh(core_axis_name="c", subcore_axis_name="s"),
    scratch_types=[
        pltpu.VMEM((BATCH,), jnp.float32),
        pltpu.VMEM((ROWS_PER_WORKER + LANES,), jnp.int32),
        pltpu.VMEM((HEAD_ROWS * CHARGE_DIM,), jnp.float32),
        pltpu.VMEM((2 * CHUNK_ELEMS,), jnp.float32),
        pltpu.SemaphoreType.DMA,
        pltpu.SemaphoreType.DMA,
        pltpu.SemaphoreType.DMA,
        pltpu.SemaphoreType.DMA((2,)),
    ],
)


def kernel(charge, segment_ids):
    seg = segment_ids.astype(jnp.int32)
    out = _sc_kernel(charge.astype(jnp.float32), seg)
    return out.reshape(TOTAL_NODES, CHARGE_DIM)


# speedup vs baseline: 1.0610x; 1.0064x over previous
import jax
import jax.numpy as jnp
from jax import lax
from jax.experimental import pallas as pl
from jax.experimental.pallas import tpu as pltpu
from jax.experimental.pallas import tpu_sc as plsc

CHARGE_DIM = 128
BATCH = 16
TOTAL_NODES = 32768

NUM_CORES = 2
NUM_SUBCORES = 16
LANES = 16
NUM_WORKERS = NUM_CORES * NUM_SUBCORES
ROWS_PER_WORKER = TOTAL_NODES // NUM_WORKERS
HEAD_ROWS = 16
CHUNK_ROWS = 112
NUM_CHUNKS = (ROWS_PER_WORKER - HEAD_ROWS) // CHUNK_ROWS
assert HEAD_ROWS + NUM_CHUNKS * CHUNK_ROWS == ROWS_PER_WORKER
COLS = CHARGE_DIM // LANES
CHUNK_ELEMS = CHUNK_ROWS * CHARGE_DIM


def _sc_body(charge_hbm, seg_hbm, out_hbm, charge_v, seg_v, bufh, buf,
             semc, sems_, semh, sem):
    wid = lax.axis_index("s") * NUM_CORES + lax.axis_index("c")
    base = wid * ROWS_PER_WORKER

    in0 = pltpu.async_copy(charge_hbm, charge_v, semc)
    in1 = pltpu.async_copy(seg_hbm.at[pl.ds(base, ROWS_PER_WORKER)],
                           seg_v.at[pl.ds(0, ROWS_PER_WORKER)], sems_)
    in0.wait()
    in1.wait()

    charge_reg = charge_v[...]
    zeros16 = jnp.zeros((LANES,), jnp.int32)

    def fast_fill(buf_, off, cval, rows):
        row = jnp.full((LANES,), cval, dtype=jnp.float32)

        def body(i, _):
            for j in range(COLS):
                buf_[pl.ds(off + i * CHARGE_DIM + j * LANES, LANES)] = row
            return ()

        lax.fori_loop(0, rows, body, (), unroll=4)

    def perrow_fill(buf_, off, row0, rows):
        def body(i, _):
            sv = seg_v[pl.ds(row0 + i, LANES)]
            sid = sv.at[zeros16].get(mode="promise_in_bounds")
            row = charge_reg.at[sid].get(mode="promise_in_bounds")
            for j in range(COLS):
                buf_[pl.ds(off + i * CHARGE_DIM + j * LANES, LANES)] = row
            return ()

        lax.fori_loop(0, rows, body, (), unroll=2)

    svh = seg_v[pl.ds(0, LANES)]
    uniform_h = svh[0] == svh[LANES - 1]
    cval_h = charge_reg.at[svh].get(mode="promise_in_bounds")[0]

    @pl.when(uniform_h)
    def _():
        fast_fill(bufh, 0, cval_h, HEAD_ROWS)

    @pl.when(jnp.logical_not(uniform_h))
    def _():
        perrow_fill(bufh, 0, 0, HEAD_ROWS)

    head_copy = pltpu.async_copy(
        bufh, out_hbm.at[pl.ds(base * CHARGE_DIM, HEAD_ROWS * CHARGE_DIM)],
        semh)

    def chunk_body(c, carry):
        valid0, val0, valid1, val1 = carry
        row0 = HEAD_ROWS + c * CHUNK_ROWS
        par = lax.rem(c, 2)
        par0 = par == 0
        off = par * CHUNK_ELEMS
        sv0 = seg_v[pl.ds(row0, LANES)]
        svl = seg_v[pl.ds(row0 + CHUNK_ROWS - LANES, LANES)]
        uniform = sv0[0] == svl[LANES - 1]
        cval = charge_reg.at[sv0].get(mode="promise_in_bounds")[0]
        same = jnp.where(par0, valid0 & (cval == val0),
                         valid1 & (cval == val1))

        @pl.when(c >= 2)
        def _():
            pltpu.make_async_copy(
                out_hbm.at[pl.ds(0, CHUNK_ELEMS)],
                buf.at[pl.ds(off, CHUNK_ELEMS)], sem.at[par]).wait()

        @pl.when(uniform & jnp.logical_not(same))
        def _():
            fast_fill(buf, off, cval, CHUNK_ROWS)

        @pl.when(jnp.logical_not(uniform))
        def _():
            perrow_fill(buf, off, row0, CHUNK_ROWS)

        pltpu.async_copy(
            buf.at[pl.ds(off, CHUNK_ELEMS)],
            out_hbm.at[pl.ds((base + row0) * CHARGE_DIM, CHUNK_ELEMS)],
            sem.at[par])

        valid0 = jnp.where(par0, uniform, valid0)
        val0 = jnp.where(par0, cval, val0)
        valid1 = jnp.where(par0, valid1, uniform)
        val1 = jnp.where(par0, val1, cval)
        return valid0, val0, valid1, val1

    lax.fori_loop(0, NUM_CHUNKS, chunk_body,
                  (jnp.bool_(False), jnp.float32(0.0),
                   jnp.bool_(False), jnp.float32(0.0)))

    pltpu.make_async_copy(
        out_hbm.at[pl.ds(0, CHUNK_ELEMS)],
        buf.at[pl.ds(0, CHUNK_ELEMS)], sem.at[0]).wait()
    pltpu.make_async_copy(
        out_hbm.at[pl.ds(0, CHUNK_ELEMS)],
        buf.at[pl.ds(CHUNK_ELEMS, CHUNK_ELEMS)], sem.at[1]).wait()
    head_copy.wait()


_sc_kernel = pl.kernel(
    _sc_body,
    out_type=jax.ShapeDtypeStruct((TOTAL_NODES * CHARGE_DIM,), jnp.float32),
    mesh=plsc.VectorSubcoreMesh(core_axis_name="c", subcore_axis_name="s"),
    scratch_types=[
        pltpu.VMEM((BATCH,), jnp.float32),
        pltpu.VMEM((ROWS_PER_WORKER + LANES,), jnp.int32),
        pltpu.VMEM((HEAD_ROWS * CHARGE_DIM,), jnp.float32),
        pltpu.VMEM((2 * CHUNK_ELEMS,), jnp.float32),
        pltpu.SemaphoreType.DMA,
        pltpu.SemaphoreType.DMA,
        pltpu.SemaphoreType.DMA,
        pltpu.SemaphoreType.DMA((2,)),
    ],
)


def kernel(charge, segment_ids):
    seg = segment_ids.astype(jnp.int32)
    out = _sc_kernel(charge.astype(jnp.float32), seg)
    return out.reshape(TOTAL_NODES, CHARGE_DIM)
